# baseline (speedup 1.0000x reference)
import jax
import jax.numpy as jnp
from jax.experimental import pallas as pl
from jax.experimental.pallas import tpu as pltpu

_N = 128
_K = 16
_OUT_DIM = 256

_SQRT_HALF = 0.7071067811865476
_EPS = 1e-5


def _gelu(x):
    return 0.5 * x * (1.0 + jax.lax.erf(x * _SQRT_HALF))


def _ln(x, g, b):
    m = jnp.mean(x, axis=-1, keepdims=True)
    c = x - m
    v = jnp.mean(c * c, axis=-1, keepdims=True)
    return c * jax.lax.rsqrt(v + _EPS) * g + b


def _net_kernel(pos_ref, x_ref, *refs):
    out_ref = refs[-1]
    prefs = refs[:-1]
    for s in range(pos_ref.shape[0]):
        _one_sample(pos_ref, x_ref, prefs, out_ref, s)


def _one_sample(pos_ref, x_ref, prefs, out_ref, s):
    posb = pos_ref[s:s + 1].reshape(_N, 8)
    p2 = posb * posb
    prod = jax.lax.dot_general(
        posb, posb, (((1,), (1,)), ((), ())),
        preferred_element_type=jnp.float32)
    sq = (p2[:, 0:1] + p2[:, 1:2]) + p2[:, 2:3]
    sq_mat = jnp.broadcast_to(sq, (_N, _N))
    dmat = (sq_mat + sq_mat.T) - 2.0 * prod

    lane = jax.lax.broadcasted_iota(jnp.int32, (_N, _N), 1)
    gs = []
    for _ in range(_K):
        m = jnp.min(dmat, axis=1, keepdims=True)
        cand = jnp.where(dmat == m, lane, _N)
        amin = jnp.min(cand, axis=1, keepdims=True)
        hit = lane == amin
        gs.append(hit.astype(jnp.float32))
        dmat = jnp.where(hit, 3.0e38, dmat)
    G = jnp.concatenate(gs, axis=0)

    h = x_ref[s:s + 1].reshape(_N, x_ref.shape[-1])
    for l in range(3):
        (wb, wd, b1, g1, be1, w2, b2, g2, be2,
         w3, b3, g3, be3, ws, bs) = prefs[l * 15:(l + 1) * 15]
        xb = jnp.dot(h, wb[...], preferred_element_type=jnp.float32, precision=jax.lax.Precision.HIGHEST)
        xc = jnp.dot(h, wd[...], preferred_element_type=jnp.float32, precision=jax.lax.Precision.HIGHEST) + b1[...]
        nb = jnp.dot(G, xb, preferred_element_type=jnp.float32, precision=jax.lax.Precision.HIGHEST)
        hh = nb + jnp.concatenate([xc] * _K, axis=0)
        hh = _gelu(_ln(hh, g1[...], be1[...]))
        hh = jnp.dot(hh, w2[...], preferred_element_type=jnp.float32, precision=jax.lax.Precision.HIGHEST) + b2[...]
        hh = _gelu(_ln(hh, g2[...], be2[...]))
        hh = jnp.dot(hh, w3[...], preferred_element_type=jnp.float32, precision=jax.lax.Precision.HIGHEST) + b3[...]
        hh = _gelu(_ln(hh, g3[...], be3[...]))
        agg = hh[0:_N]
        for j in range(1, _K):
            agg = jnp.maximum(agg, hh[j * _N:(j + 1) * _N])
        sc = jnp.dot(h, ws[...], preferred_element_type=jnp.float32, precision=jax.lax.Precision.HIGHEST) + bs[...]
        h = _gelu(agg + sc)
    out_ref[s:s + 1] = h.reshape(1, _N, _OUT_DIM)


def _split(p):
    W1 = p['W1']
    d = W1.shape[0] // 2
    wb = W1[d:]
    wd = W1[:d] - W1[d:]
    r = lambda a: a.reshape(1, -1)
    return [wb, wd, r(p['b1']), r(p['g1']), r(p['be1']),
            p['W2'], r(p['b2']), r(p['g2']), r(p['be2']),
            p['W3'], r(p['b3']), r(p['g3']), r(p['be3']),
            p['Ws'], r(p['bs'])]


@jax.jit
def _run(x, pos, params):
    B = x.shape[0]
    posb = jnp.pad(pos, ((0, 0), (0, 0), (0, 8 - pos.shape[-1])))
    flat = []
    for p in params:
        flat += _split(p)
    sb = 2
    in_specs = [
        pl.BlockSpec((sb, _N, 8), lambda b: (b, 0, 0)),
        pl.BlockSpec((sb, _N, x.shape[-1]), lambda b: (b, 0, 0)),
    ]
    for a in flat:
        in_specs.append(
            pl.BlockSpec(a.shape, lambda b, nd=a.ndim: (0,) * nd))
    return pl.pallas_call(
        _net_kernel,
        grid=(B // sb,),
        in_specs=in_specs,
        out_specs=pl.BlockSpec((sb, _N, _OUT_DIM), lambda b: (b, 0, 0)),
        out_shape=jax.ShapeDtypeStruct((B, _N, _OUT_DIM), jnp.float32),
        compiler_params=pltpu.CompilerParams(
            dimension_semantics=("parallel",)),
    )(posb, x, *flat)


def kernel(x, pos, params, ks):
    return _run(x, pos, params)


# speedup vs baseline: 1.3069x; 1.3069x over previous
import jax
import jax.numpy as jnp
from jax.experimental import pallas as pl
from jax.experimental.pallas import tpu as pltpu

_N = 128
_K = 16
_OUT_DIM = 256

_SQRT_HALF = 0.7071067811865476
_EPS = 1e-5


def _gelu(x):
    return 0.5 * x * (1.0 + jax.lax.erf(x * _SQRT_HALF))


def _ln(x, g, b):
    m = jnp.mean(x, axis=-1, keepdims=True)
    c = x - m
    v = jnp.mean(c * c, axis=-1, keepdims=True)
    return c * jax.lax.rsqrt(v + _EPS) * g + b


def _net_kernel(pos_ref, x_ref, *refs):
    out_ref = refs[-1]
    prefs = refs[:-1]
    for s in range(pos_ref.shape[0]):
        _one_sample(pos_ref, x_ref, prefs, out_ref, s)


def _one_sample(pos_ref, x_ref, prefs, out_ref, s):
    posb = pos_ref[s:s + 1].reshape(_N, 8)
    p2 = posb * posb
    prod = jax.lax.dot_general(
        posb, posb, (((1,), (1,)), ((), ())),
        preferred_element_type=jnp.float32)
    sq = (p2[:, 0:1] + p2[:, 1:2]) + p2[:, 2:3]
    sq_mat = jnp.broadcast_to(sq, (_N, _N))
    dmat = (sq_mat + sq_mat.T) - 2.0 * prod

    lane = jax.lax.broadcasted_iota(jnp.int32, (_N, _N), 1)
    gs = []
    for _ in range(_K):
        m = jnp.min(dmat, axis=1, keepdims=True)
        cand = jnp.where(dmat == m, lane, _N)
        amin = jnp.min(cand, axis=1, keepdims=True)
        hit = lane == amin
        gs.append(hit.astype(jnp.bfloat16))
        dmat = jnp.where(hit, 3.0e38, dmat)
    G = jnp.concatenate(gs, axis=0)

    h = x_ref[s:s + 1].reshape(_N, x_ref.shape[-1])
    for l in range(3):
        (wb, wd, b1, g1, be1, w2, b2, g2, be2,
         w3, b3, g3, be3, ws, bs) = prefs[l * 15:(l + 1) * 15]
        xb = jnp.dot(h, wb[...], preferred_element_type=jnp.float32, precision=jax.lax.Precision.HIGHEST)
        xc = jnp.dot(h, wd[...], preferred_element_type=jnp.float32, precision=jax.lax.Precision.HIGHEST) + b1[...]
        hi = xb.astype(jnp.bfloat16)
        r1 = xb - hi.astype(jnp.float32)
        md = r1.astype(jnp.bfloat16)
        lo = (r1 - md.astype(jnp.float32)).astype(jnp.bfloat16)
        _gdot = lambda a: jax.lax.dot_general(
            G, a, (((1,), (0,)), ((), ())),
            preferred_element_type=jnp.float32)
        nb = (_gdot(hi) + _gdot(md)) + _gdot(lo)
        hh = nb + jnp.concatenate([xc] * _K, axis=0)
        hh = _gelu(_ln(hh, g1[...], be1[...]))
        hh = jnp.dot(hh, w2[...], preferred_element_type=jnp.float32, precision=jax.lax.Precision.HIGHEST) + b2[...]
        hh = _gelu(_ln(hh, g2[...], be2[...]))
        hh = jnp.dot(hh, w3[...], preferred_element_type=jnp.float32, precision=jax.lax.Precision.HIGHEST) + b3[...]
        hh = _gelu(_ln(hh, g3[...], be3[...]))
        agg = hh[0:_N]
        for j in range(1, _K):
            agg = jnp.maximum(agg, hh[j * _N:(j + 1) * _N])
        sc = jnp.dot(h, ws[...], preferred_element_type=jnp.float32, precision=jax.lax.Precision.HIGHEST) + bs[...]
        h = _gelu(agg + sc)
    out_ref[s:s + 1] = h.reshape(1, _N, _OUT_DIM)


def _split(p):
    W1 = p['W1']
    d = W1.shape[0] // 2
    wb = W1[d:]
    wd = W1[:d] - W1[d:]
    r = lambda a: a.reshape(1, -1)
    return [wb, wd, r(p['b1']), r(p['g1']), r(p['be1']),
            p['W2'], r(p['b2']), r(p['g2']), r(p['be2']),
            p['W3'], r(p['b3']), r(p['g3']), r(p['be3']),
            p['Ws'], r(p['bs'])]


@jax.jit
def _run(x, pos, params):
    B = x.shape[0]
    posb = jnp.pad(pos, ((0, 0), (0, 0), (0, 8 - pos.shape[-1])))
    flat = []
    for p in params:
        flat += _split(p)
    sb = 1
    in_specs = [
        pl.BlockSpec((sb, _N, 8), lambda b: (b, 0, 0)),
        pl.BlockSpec((sb, _N, x.shape[-1]), lambda b: (b, 0, 0)),
    ]
    for a in flat:
        in_specs.append(
            pl.BlockSpec(a.shape, lambda b, nd=a.ndim: (0,) * nd))
    return pl.pallas_call(
        _net_kernel,
        grid=(B // sb,),
        in_specs=in_specs,
        out_specs=pl.BlockSpec((sb, _N, _OUT_DIM), lambda b: (b, 0, 0)),
        out_shape=jax.ShapeDtypeStruct((B, _N, _OUT_DIM), jnp.float32),
        compiler_params=pltpu.CompilerParams(
            dimension_semantics=("arbitrary",)),
    )(posb, x, *flat)


def kernel(x, pos, params, ks):
    return _run(x, pos, params)


# speedup vs baseline: 1.3823x; 1.0577x over previous
import jax
import jax.numpy as jnp
from jax.experimental import pallas as pl
from jax.experimental.pallas import tpu as pltpu

_N = 128
_K = 16
_OUT_DIM = 256

_SQRT_HALF = 0.7071067811865476
_EPS = 1e-5


def _gelu(x):
    return 0.5 * x * (1.0 + jax.lax.erf(x * _SQRT_HALF))


def _ln(x, g, b):
    m = jnp.mean(x, axis=-1, keepdims=True)
    c = x - m
    v = jnp.mean(c * c, axis=-1, keepdims=True)
    return c * jax.lax.rsqrt(v + _EPS) * g + b


def _net_kernel(pos_ref, x_ref, *refs):
    out_ref = refs[-1]
    prefs = refs[:-1]
    for s in range(pos_ref.shape[0]):
        _one_sample(pos_ref, x_ref, prefs, out_ref, s)


def _one_sample(pos_ref, x_ref, prefs, out_ref, s):
    posb = pos_ref[s:s + 1].reshape(_N, 8)
    p2 = posb * posb
    prod = jax.lax.dot_general(
        posb, posb, (((1,), (1,)), ((), ())),
        preferred_element_type=jnp.float32)
    sq = (p2[:, 0:1] + p2[:, 1:2]) + p2[:, 2:3]
    sq_mat = jnp.broadcast_to(sq, (_N, _N))
    dmat = (sq_mat + sq_mat.T) - 2.0 * prod

    lane = jax.lax.broadcasted_iota(jnp.int32, (_N, _N), 1)
    gs = []
    for _ in range(_K):
        m = jnp.min(dmat, axis=1, keepdims=True)
        cand = jnp.where(dmat == m, lane, _N)
        amin = jnp.min(cand, axis=1, keepdims=True)
        hit = lane == amin
        gs.append(hit.astype(jnp.float32))
        dmat = jnp.where(hit, 3.0e38, dmat)
    Gb = jnp.concatenate(gs, axis=0).astype(jnp.bfloat16)
    G3 = jnp.concatenate([Gb, Gb, Gb], axis=1)

    h = x_ref[s:s + 1].reshape(_N, x_ref.shape[-1])
    for l in range(3):
        (wb, wd, b1, g1, be1, w2, b2, g2, be2,
         w3, b3, g3, be3, ws, bs) = prefs[l * 15:(l + 1) * 15]
        xb = jnp.dot(h, wb[...], preferred_element_type=jnp.float32, precision=jax.lax.Precision.HIGHEST)
        xc = jnp.dot(h, wd[...], preferred_element_type=jnp.float32, precision=jax.lax.Precision.HIGHEST) + b1[...]
        hi = xb.astype(jnp.bfloat16)
        r1 = xb - hi.astype(jnp.float32)
        md = r1.astype(jnp.bfloat16)
        lo = (r1 - md.astype(jnp.float32)).astype(jnp.bfloat16)
        A = jnp.concatenate([hi, md, lo], axis=0)
        nb = jax.lax.dot_general(
            G3, A, (((1,), (0,)), ((), ())),
            preferred_element_type=jnp.float32)
        hh = nb + jnp.concatenate([xc] * _K, axis=0)
        hh = _gelu(_ln(hh, g1[...], be1[...]))
        hh = jnp.dot(hh, w2[...], preferred_element_type=jnp.float32, precision=jax.lax.Precision.HIGHEST) + b2[...]
        hh = _gelu(_ln(hh, g2[...], be2[...]))
        hh = jnp.dot(hh, w3[...], preferred_element_type=jnp.float32, precision=jax.lax.Precision.HIGHEST) + b3[...]
        hh = _gelu(_ln(hh, g3[...], be3[...]))
        agg = hh[0:_N]
        for j in range(1, _K):
            agg = jnp.maximum(agg, hh[j * _N:(j + 1) * _N])
        sc = jnp.dot(h, ws[...], preferred_element_type=jnp.float32, precision=jax.lax.Precision.HIGHEST) + bs[...]
        h = _gelu(agg + sc)
    out_ref[s:s + 1] = h.reshape(1, _N, _OUT_DIM)


def _split(p):
    W1 = p['W1']
    d = W1.shape[0] // 2
    wb = W1[d:]
    wd = W1[:d] - W1[d:]
    r = lambda a: a.reshape(1, -1)
    return [wb, wd, r(p['b1']), r(p['g1']), r(p['be1']),
            p['W2'], r(p['b2']), r(p['g2']), r(p['be2']),
            p['W3'], r(p['b3']), r(p['g3']), r(p['be3']),
            p['Ws'], r(p['bs'])]


@jax.jit
def _run(x, pos, params):
    B = x.shape[0]
    posb = jnp.pad(pos, ((0, 0), (0, 0), (0, 8 - pos.shape[-1])))
    flat = []
    for p in params:
        flat += _split(p)
    sb = 1
    in_specs = [
        pl.BlockSpec((sb, _N, 8), lambda b: (b, 0, 0)),
        pl.BlockSpec((sb, _N, x.shape[-1]), lambda b: (b, 0, 0)),
    ]
    for a in flat:
        in_specs.append(
            pl.BlockSpec(a.shape, lambda b, nd=a.ndim: (0,) * nd))
    return pl.pallas_call(
        _net_kernel,
        grid=(B // sb,),
        in_specs=in_specs,
        out_specs=pl.BlockSpec((sb, _N, _OUT_DIM), lambda b: (b, 0, 0)),
        out_shape=jax.ShapeDtypeStruct((B, _N, _OUT_DIM), jnp.float32),
        compiler_params=pltpu.CompilerParams(
            dimension_semantics=("arbitrary",)),
    )(posb, x, *flat)


def kernel(x, pos, params, ks):
    return _run(x, pos, params)
